# baseline (speedup 1.0000x reference)
import functools

import jax
import jax.numpy as jnp
from jax import lax
from jax.experimental import pallas as pl
from jax.experimental.pallas import tpu as pltpu
from jax.experimental.pallas import tpu_sc as plsc

NUM_CLASSES = 1000000
EMBEDDING_DIM = 64
BATCH = 16384

_info = plsc.get_sparse_core_info()
_NUM_WORKERS = _info.num_cores * _info.num_subcores
_B_PER_W = BATCH // _NUM_WORKERS

_CHUNK = 128
_N_CHUNKS = _B_PER_W // _CHUNK

_PAIR = 4096
_PAIR_LOG2 = 12
_NBLK = 72
_SPLIT = _NBLK * 2 * _PAIR
_SCRATCH_ROWS = _NBLK * _PAIR


def _transpose_body(a_ref, b_ref, o_ref):
    o_ref[:, 0:EMBEDDING_DIM] = a_ref[...].T
    o_ref[:, EMBEDDING_DIM:128] = b_ref[...].T


def _relayout(table_t):
    return pl.pallas_call(
        _transpose_body,
        grid=(_NBLK,),
        in_specs=[
            pl.BlockSpec((EMBEDDING_DIM, _PAIR), lambda i: (0, 2 * i)),
            pl.BlockSpec((EMBEDDING_DIM, _PAIR), lambda i: (0, 2 * i + 1)),
        ],
        out_specs=pl.BlockSpec((_PAIR, 128), lambda i: (i, 0)),
        out_shape=jax.ShapeDtypeStruct((_SCRATCH_ROWS, 128), jnp.float32),
    )(table_t, table_t)


def _high_rows(labels, table):
    mesh = plsc.VectorSubcoreMesh(core_axis_name="c", subcore_axis_name="s")

    @functools.partial(
        pl.kernel,
        mesh=mesh,
        out_type=jax.ShapeDtypeStruct((BATCH, EMBEDDING_DIM), jnp.float32),
        scratch_types=[
            pltpu.VMEM((_B_PER_W,), jnp.int32),
            pltpu.VMEM((_B_PER_W, EMBEDDING_DIM), jnp.float32),
            pltpu.SemaphoreType.DMA,
            pltpu.SemaphoreType.DMA,
        ],
        compiler_params=pltpu.CompilerParams(needs_layout_passes=False),
    )
    def a(labels_hbm, table_hbm, out_hbm, idx_v, rows_v, gsem, ssem):
        wid = lax.axis_index("s") * _info.num_cores + lax.axis_index("c")
        base = wid * _B_PER_W
        pltpu.sync_copy(labels_hbm.at[pl.ds(base, _B_PER_W)], idx_v)
        nfired = jnp.int32(0)
        for v16 in range(_B_PER_W // 16):
            jb = v16 * 16
            vec = idx_v[pl.ds(jb, 16)]
            qual = jnp.where(vec >= _SPLIT, jnp.int32(1), jnp.int32(0))
            nfired = nfired + lax.reduce_sum_p.bind(qual, axes=(0,))
            for u in range(16):
                r = vec[u]

                @pl.when(r >= _SPLIT)
                def _(r=r, j=jb + u):
                    pltpu.async_copy(
                        table_hbm.at[pl.ds(r, 1)],
                        rows_v.at[pl.ds(j, 1)],
                        gsem,
                    )

        def drain_one(i, carry):
            pltpu.make_async_copy(
                table_hbm.at[pl.ds(0, 1)], rows_v.at[pl.ds(0, 1)], gsem
            ).wait()
            return carry

        lax.fori_loop(0, nfired, drain_one, jnp.int32(0))
        pltpu.sync_copy(rows_v, out_hbm.at[pl.ds(base, _B_PER_W)])

    return a(labels, table)


def _gather(labels, scratch, high):
    mesh = plsc.VectorSubcoreMesh(core_axis_name="c", subcore_axis_name="s")

    @functools.partial(
        pl.kernel,
        mesh=mesh,
        out_type=jax.ShapeDtypeStruct((BATCH, EMBEDDING_DIM), jnp.float32),
        scratch_types=[
            pltpu.VMEM((_B_PER_W,), jnp.int32),
            pltpu.VMEM((_B_PER_W,), jnp.int32),
            pltpu.VMEM((_B_PER_W,), jnp.int32),
            pltpu.VMEM((_CHUNK, 128), jnp.float32),
            pltpu.VMEM((_CHUNK, 128), jnp.float32),
            pltpu.VMEM((_CHUNK, EMBEDDING_DIM), jnp.float32),
            pltpu.VMEM((_CHUNK, EMBEDDING_DIM), jnp.float32),
            pltpu.VMEM((_CHUNK, EMBEDDING_DIM), jnp.float32),
            pltpu.VMEM((_CHUNK, EMBEDDING_DIM), jnp.float32),
            pltpu.SemaphoreType.DMA((_N_CHUNKS,)),
            pltpu.SemaphoreType.DMA((_N_CHUNKS,)),
            pltpu.SemaphoreType.DMA,
        ],
        compiler_params=pltpu.CompilerParams(needs_layout_passes=False),
    )
    def c(labels_hbm, scratch_hbm, high_hbm, out_hbm, idx_v, tidx_v, half_v,
          rows_a, rows_b, high_a, high_b, out_a, out_b, gsems, hsems, ssem):
        wid = lax.axis_index("s") * _info.num_cores + lax.axis_index("c")
        base = wid * _B_PER_W
        pltpu.sync_copy(labels_hbm.at[pl.ds(base, _B_PER_W)], idx_v)
        for i in range(_B_PER_W // 16):
            l = idx_v[pl.ds(i * 16, 16)]
            r_main = lax.bitwise_or(
                lax.shift_left(
                    lax.shift_right_logical(l, _PAIR_LOG2 + 1), _PAIR_LOG2
                ),
                lax.bitwise_and(l, jnp.int32(_PAIR - 1)),
            )
            h_main = lax.bitwise_and(
                lax.shift_right_logical(l, _PAIR_LOG2), jnp.int32(1)
            )
            in_high = l >= _SPLIT
            tidx_v[pl.ds(i * 16, 16)] = jnp.where(in_high, jnp.int32(0), r_main)
            half_v[pl.ds(i * 16, 16)] = jnp.where(
                in_high, jnp.int32(128), h_main * EMBEDDING_DIM
            )

        rows_bufs = [rows_a, rows_b]
        high_bufs = [high_a, high_b]
        out_bufs = [out_a, out_b]
        gathers = {}
        hcopies = {}

        def fire(ch):
            gathers[ch] = pltpu.async_copy(
                scratch_hbm.at[tidx_v.at[pl.ds(ch * _CHUNK, _CHUNK)]],
                rows_bufs[ch % 2],
                gsems.at[ch],
            )
            hcopies[ch] = pltpu.async_copy(
                high_hbm.at[pl.ds(base + ch * _CHUNK, _CHUNK)],
                high_bufs[ch % 2],
                hsems.at[ch],
            )

        fire(0)
        fire(1)
        writebacks = []
        for ch in range(_N_CHUNKS):
            cbase = ch * _CHUNK
            rows_v = rows_bufs[ch % 2]
            high_v = high_bufs[ch % 2]
            out_v = out_bufs[ch % 2]
            gathers[ch].wait()
            hcopies[ch].wait()
            if ch >= 2:
                writebacks[ch - 2].wait()
            ebs = [
                half_v[pl.ds(cbase + jb * 16, 16)]
                for jb in range(_CHUNK // 16)
            ]

            def colbody(cc, carry, rows_v=rows_v, high_v=high_v, out_v=out_v,
                        ebs=ebs):
                cvec = jnp.zeros((16,), jnp.int32) + cc
                for jb in range(_CHUNK // 16):
                    j_vec = lax.iota(jnp.int32, 16) + jb * 16
                    take_a = ebs[jb] >= 128
                    e_vec = jnp.where(take_a, jnp.int32(0), ebs[jb]) + cvec
                    v_s = plsc.load_gather(rows_v, [j_vec, e_vec])
                    v_a = plsc.load_gather(high_v, [j_vec, cvec])
                    vals = jnp.where(take_a, v_a, v_s)
                    plsc.store_scatter(out_v, [j_vec, cvec], vals)
                return carry

            lax.fori_loop(0, EMBEDDING_DIM, colbody, 0)
            if ch + 2 < _N_CHUNKS:
                fire(ch + 2)
            writebacks.append(
                pltpu.async_copy(
                    out_v,
                    out_hbm.at[pl.ds(base + cbase, _CHUNK)],
                    ssem,
                )
            )
        for wb in writebacks[-2:]:
            wb.wait()

    return c(labels, scratch, high)


def kernel(labels, table):
    labels = labels.astype(jnp.int32)
    high = _high_rows(labels, table)
    scratch = _relayout(table.T)
    return _gather(labels, scratch, high)


# speedup vs baseline: 2.4635x; 2.4635x over previous
import functools

import jax
import jax.numpy as jnp
from jax import lax
from jax.experimental import pallas as pl
from jax.experimental.pallas import tpu as pltpu
from jax.experimental.pallas import tpu_sc as plsc

NUM_CLASSES = 1000000
EMBEDDING_DIM = 64
BATCH = 16384

_info = plsc.get_sparse_core_info()
_NUM_WORKERS = _info.num_cores * _info.num_subcores
_B_PER_W = BATCH // _NUM_WORKERS

_CHUNK = 128
_N_CHUNKS = _B_PER_W // _CHUNK

_PAIR = 4096
_PAIR_LOG2 = 12
_NBLK = NUM_CLASSES // (2 * _PAIR)
_MAIN = _NBLK * 2 * _PAIR
_MAIN_ROWS = _NBLK * _PAIR
_TAIL_BLKS = 5
_SCRATCH_ROWS = _MAIN_ROWS + _TAIL_BLKS * 64


def _mxu_t(x):
    hi = x.astype(jnp.bfloat16)
    lo = (x - hi.astype(jnp.float32)).astype(jnp.bfloat16)
    eye = (
        lax.broadcasted_iota(jnp.int32, (EMBEDDING_DIM, EMBEDDING_DIM), 0)
        == lax.broadcasted_iota(jnp.int32, (EMBEDDING_DIM, EMBEDDING_DIM), 1)
    ).astype(jnp.bfloat16)
    dn = (((0,), (0,)), ((), ()))
    return lax.dot_general(
        hi, eye, dn, preferred_element_type=jnp.float32
    ) + lax.dot_general(lo, eye, dn, preferred_element_type=jnp.float32)


def _transpose_body(a_ref, b_ref, o_ref):
    o_ref[:, 0:EMBEDDING_DIM] = _mxu_t(a_ref[...])
    o_ref[:, EMBEDDING_DIM:128] = _mxu_t(b_ref[...])


def _tail_body(x_ref, s_ref, o_ref):
    del s_ref
    o_ref[:, 0:EMBEDDING_DIM] = _mxu_t(x_ref[:, 0:EMBEDDING_DIM])
    o_ref[:, EMBEDDING_DIM:128] = _mxu_t(x_ref[:, EMBEDDING_DIM:128])


def _relayout(table_t):
    main = pl.pallas_call(
        _transpose_body,
        grid=(_NBLK,),
        in_specs=[
            pl.BlockSpec((EMBEDDING_DIM, _PAIR), lambda i: (0, 2 * i)),
            pl.BlockSpec((EMBEDDING_DIM, _PAIR), lambda i: (0, 2 * i + 1)),
        ],
        out_specs=pl.BlockSpec((_PAIR, 128), lambda i: (i, 0)),
        out_shape=jax.ShapeDtypeStruct((_SCRATCH_ROWS, 128), jnp.float32),
    )(table_t, table_t)
    return pl.pallas_call(
        _tail_body,
        grid=(_TAIL_BLKS,),
        in_specs=[
            pl.BlockSpec((EMBEDDING_DIM, 128), lambda i: (0, _MAIN // 128 + i)),
            pl.BlockSpec(memory_space=pl.ANY),
        ],
        out_specs=pl.BlockSpec((64, 128), lambda i: (_MAIN_ROWS // 64 + i, 0)),
        out_shape=jax.ShapeDtypeStruct((_SCRATCH_ROWS, 128), jnp.float32),
        input_output_aliases={1: 0},
    )(table_t, main)


def _gather(labels, scratch):
    mesh = plsc.VectorSubcoreMesh(core_axis_name="c", subcore_axis_name="s")

    @functools.partial(
        pl.kernel,
        mesh=mesh,
        out_type=jax.ShapeDtypeStruct((BATCH, EMBEDDING_DIM), jnp.float32),
        scratch_types=[
            pltpu.VMEM((_B_PER_W,), jnp.int32),
            pltpu.VMEM((_B_PER_W,), jnp.int32),
            pltpu.VMEM((_B_PER_W,), jnp.int32),
            pltpu.VMEM((_CHUNK, 128), jnp.float32),
            pltpu.VMEM((_CHUNK, 128), jnp.float32),
            pltpu.VMEM((_CHUNK, EMBEDDING_DIM), jnp.float32),
            pltpu.VMEM((_CHUNK, EMBEDDING_DIM), jnp.float32),
            pltpu.SemaphoreType.DMA((_N_CHUNKS,)),
            pltpu.SemaphoreType.DMA,
        ],
        compiler_params=pltpu.CompilerParams(needs_layout_passes=False),
    )
    def c(labels_hbm, scratch_hbm, out_hbm, idx_v, tidx_v, half_v, rows_a,
          rows_b, out_a, out_b, gsems, ssem):
        wid = lax.axis_index("s") * _info.num_cores + lax.axis_index("c")
        base = wid * _B_PER_W
        pltpu.sync_copy(labels_hbm.at[pl.ds(base, _B_PER_W)], idx_v)
        for i in range(_B_PER_W // 16):
            l = idx_v[pl.ds(i * 16, 16)]
            r_main = lax.bitwise_or(
                lax.shift_left(
                    lax.shift_right_logical(l, _PAIR_LOG2 + 1), _PAIR_LOG2
                ),
                lax.bitwise_and(l, jnp.int32(_PAIR - 1)),
            )
            h_main = lax.bitwise_and(
                lax.shift_right_logical(l, _PAIR_LOG2), jnp.int32(1)
            )
            t = l - _MAIN
            r_tail = (
                _MAIN_ROWS
                + lax.shift_left(lax.shift_right_logical(t, 7), 6)
                + lax.bitwise_and(t, jnp.int32(63))
            )
            h_tail = lax.bitwise_and(
                lax.shift_right_logical(t, 6), jnp.int32(1)
            )
            in_tail = l >= _MAIN
            tidx_v[pl.ds(i * 16, 16)] = jnp.where(in_tail, r_tail, r_main)
            half_v[pl.ds(i * 16, 16)] = (
                jnp.where(in_tail, h_tail, h_main) * EMBEDDING_DIM
            )

        rows_bufs = [rows_a, rows_b]
        out_bufs = [out_a, out_b]
        gathers = {}

        def fire(ch):
            gathers[ch] = pltpu.async_copy(
                scratch_hbm.at[tidx_v.at[pl.ds(ch * _CHUNK, _CHUNK)]],
                rows_bufs[ch % 2],
                gsems.at[ch],
            )

        fire(0)
        fire(1)
        writebacks = []
        for ch in range(_N_CHUNKS):
            cbase = ch * _CHUNK
            rows_v = rows_bufs[ch % 2]
            out_v = out_bufs[ch % 2]
            gathers[ch].wait()
            if ch >= 2:
                writebacks[ch - 2].wait()
            ebs = [
                half_v[pl.ds(cbase + jb * 16, 16)]
                for jb in range(_CHUNK // 16)
            ]

            def colbody(cc, carry, rows_v=rows_v, out_v=out_v, ebs=ebs):
                cvec = jnp.zeros((16,), jnp.int32) + cc
                for jb in range(_CHUNK // 16):
                    j_vec = lax.iota(jnp.int32, 16) + jb * 16
                    vals = plsc.load_gather(rows_v, [j_vec, ebs[jb] + cvec])
                    plsc.store_scatter(out_v, [j_vec, cvec], vals)
                return carry

            lax.fori_loop(0, EMBEDDING_DIM, colbody, 0)
            if ch + 2 < _N_CHUNKS:
                fire(ch + 2)
            writebacks.append(
                pltpu.async_copy(
                    out_v,
                    out_hbm.at[pl.ds(base + cbase, _CHUNK)],
                    ssem,
                )
            )
        for wb in writebacks[-2:]:
            wb.wait()

    return c(labels, scratch)


def kernel(labels, table):
    scratch = _relayout(table.T)
    return _gather(labels.astype(jnp.int32), scratch)


# speedup vs baseline: 2.7936x; 1.1340x over previous
import functools

import jax
import jax.numpy as jnp
from jax import lax
from jax.experimental import pallas as pl
from jax.experimental.pallas import tpu as pltpu
from jax.experimental.pallas import tpu_sc as plsc

NUM_CLASSES = 1000000
EMBEDDING_DIM = 64
BATCH = 16384

_info = plsc.get_sparse_core_info()
_NUM_WORKERS = _info.num_cores * _info.num_subcores
_B_PER_W = BATCH // _NUM_WORKERS

_CHUNK = 128
_N_CHUNKS = _B_PER_W // _CHUNK

_PAIR = 8192
_PAIR_LOG2 = 13
_NBLK = NUM_CLASSES // (2 * _PAIR)
_MAIN = _NBLK * 2 * _PAIR
_MAIN_ROWS = _NBLK * _PAIR
_TAIL_BLKS = 5
_SCRATCH_ROWS = _MAIN_ROWS + _TAIL_BLKS * 64


def _transpose_body(a_ref, b_ref, o_ref):
    o_ref[:, 0:EMBEDDING_DIM] = a_ref[...].T
    o_ref[:, EMBEDDING_DIM:128] = b_ref[...].T


def _tail_body(x_ref, s_ref, o_ref):
    del s_ref
    o_ref[:, 0:EMBEDDING_DIM] = x_ref[:, 0:EMBEDDING_DIM].T
    o_ref[:, EMBEDDING_DIM:128] = x_ref[:, EMBEDDING_DIM:128].T


def _relayout(table_t):
    main = pl.pallas_call(
        _transpose_body,
        grid=(_NBLK,),
        in_specs=[
            pl.BlockSpec((EMBEDDING_DIM, _PAIR), lambda i: (0, 2 * i)),
            pl.BlockSpec((EMBEDDING_DIM, _PAIR), lambda i: (0, 2 * i + 1)),
        ],
        out_specs=pl.BlockSpec((_PAIR, 128), lambda i: (i, 0)),
        out_shape=jax.ShapeDtypeStruct((_SCRATCH_ROWS, 128), jnp.float32),
    )(table_t, table_t)
    return pl.pallas_call(
        _tail_body,
        grid=(_TAIL_BLKS,),
        in_specs=[
            pl.BlockSpec((EMBEDDING_DIM, 128), lambda i: (0, _MAIN // 128 + i)),
            pl.BlockSpec(memory_space=pl.ANY),
        ],
        out_specs=pl.BlockSpec((64, 128), lambda i: (_MAIN_ROWS // 64 + i, 0)),
        out_shape=jax.ShapeDtypeStruct((_SCRATCH_ROWS, 128), jnp.float32),
        input_output_aliases={1: 0},
    )(table_t, main)


def _gather(labels, scratch):
    mesh = plsc.VectorSubcoreMesh(core_axis_name="c", subcore_axis_name="s")

    @functools.partial(
        pl.kernel,
        mesh=mesh,
        out_type=jax.ShapeDtypeStruct((BATCH, EMBEDDING_DIM), jnp.float32),
        scratch_types=[
            pltpu.VMEM((_B_PER_W,), jnp.int32),
            pltpu.VMEM((_B_PER_W,), jnp.int32),
            pltpu.VMEM((_B_PER_W,), jnp.int32),
            pltpu.VMEM((_CHUNK, 128), jnp.float32),
            pltpu.VMEM((_CHUNK, 128), jnp.float32),
            pltpu.VMEM((_CHUNK, EMBEDDING_DIM), jnp.float32),
            pltpu.VMEM((_CHUNK, EMBEDDING_DIM), jnp.float32),
            pltpu.SemaphoreType.DMA((_N_CHUNKS,)),
            pltpu.SemaphoreType.DMA,
        ],
        compiler_params=pltpu.CompilerParams(needs_layout_passes=False),
    )
    def c(labels_hbm, scratch_hbm, out_hbm, idx_v, tidx_v, half_v, rows_a,
          rows_b, out_a, out_b, gsems, ssem):
        wid = lax.axis_index("s") * _info.num_cores + lax.axis_index("c")
        base = wid * _B_PER_W
        pltpu.sync_copy(labels_hbm.at[pl.ds(base, _B_PER_W)], idx_v)
        for i in range(_B_PER_W // 16):
            l = idx_v[pl.ds(i * 16, 16)]
            r_main = lax.bitwise_or(
                lax.shift_left(
                    lax.shift_right_logical(l, _PAIR_LOG2 + 1), _PAIR_LOG2
                ),
                lax.bitwise_and(l, jnp.int32(_PAIR - 1)),
            )
            h_main = lax.bitwise_and(
                lax.shift_right_logical(l, _PAIR_LOG2), jnp.int32(1)
            )
            t = l - _MAIN
            r_tail = (
                _MAIN_ROWS
                + lax.shift_left(lax.shift_right_logical(t, 7), 6)
                + lax.bitwise_and(t, jnp.int32(63))
            )
            h_tail = lax.bitwise_and(
                lax.shift_right_logical(t, 6), jnp.int32(1)
            )
            in_tail = l >= _MAIN
            tidx_v[pl.ds(i * 16, 16)] = jnp.where(in_tail, r_tail, r_main)
            half_v[pl.ds(i * 16, 16)] = (
                jnp.where(in_tail, h_tail, h_main) * EMBEDDING_DIM
            )

        rows_bufs = [rows_a, rows_b]
        out_bufs = [out_a, out_b]
        gathers = {}

        def fire(ch):
            gathers[ch] = pltpu.async_copy(
                scratch_hbm.at[tidx_v.at[pl.ds(ch * _CHUNK, _CHUNK)]],
                rows_bufs[ch % 2],
                gsems.at[ch],
            )

        fire(0)
        fire(1)
        writebacks = []
        for ch in range(_N_CHUNKS):
            cbase = ch * _CHUNK
            rows_v = rows_bufs[ch % 2]
            out_v = out_bufs[ch % 2]
            gathers[ch].wait()
            if ch >= 2:
                writebacks[ch - 2].wait()
            ebs = [
                half_v[pl.ds(cbase + jb * 16, 16)]
                for jb in range(_CHUNK // 16)
            ]

            def colbody(cc, carry, rows_v=rows_v, out_v=out_v, ebs=ebs):
                cvec = jnp.zeros((16,), jnp.int32) + cc
                for jb in range(_CHUNK // 16):
                    j_vec = lax.iota(jnp.int32, 16) + jb * 16
                    vals = plsc.load_gather(rows_v, [j_vec, ebs[jb] + cvec])
                    plsc.store_scatter(out_v, [j_vec, cvec], vals)
                return carry

            lax.fori_loop(0, EMBEDDING_DIM, colbody, 0)
            if ch + 2 < _N_CHUNKS:
                fire(ch + 2)
            writebacks.append(
                pltpu.async_copy(
                    out_v,
                    out_hbm.at[pl.ds(base + cbase, _CHUNK)],
                    ssem,
                )
            )
        for wb in writebacks[-2:]:
            wb.wait()

    return c(labels, scratch)


def kernel(labels, table):
    scratch = _relayout(table.T)
    return _gather(labels.astype(jnp.int32), scratch)
